# Initial kernel scaffold; baseline (speedup 1.0000x reference)
import jax
import jax.numpy as jnp
from jax.experimental import pallas as pl


def kernel(x, adj, W, a):
    raise NotImplementedError("write your pallas kernel here")


# speedup vs baseline: 1.9066x; 1.9066x over previous
import functools

import jax
import jax.numpy as jnp
from jax.experimental import pallas as pl
from jax.experimental.pallas import tpu as pltpu

ALPHA = 0.2


def _leaky(z):
    return jnp.where(z > 0, z, ALPHA * z)


def _proj_kernel(x_ref, wcat_ref, a1_ref, a2_ref, wh_ref, f1_ref, f2_ref,
                 *, n_valid, bm):
    i = pl.program_id(0)
    wh = jnp.dot(x_ref[...], wcat_ref[...], preferred_element_type=jnp.float32)
    row = i * bm + jax.lax.broadcasted_iota(jnp.int32, (bm, 1), 0)
    wh = jnp.where(row < n_valid, wh, 0.0)
    wh_ref[...] = wh
    f1_ref[...] = jnp.dot(wh, a1_ref[...], preferred_element_type=jnp.float32)
    f2_ref[...] = jnp.dot(wh, a2_ref[...], preferred_element_type=jnp.float32)


def _att_kernel(f1_ref, m_ref, f2t_ref, whmean_ref, adj_ref, wh_ref,
                out_ref, den_ref, *, n_valid, bc, nj, nheads, od):
    j = pl.program_id(1)

    @pl.when(j == 0)
    def _init():
        out_ref[...] = jnp.zeros_like(out_ref)
        den_ref[...] = jnp.zeros_like(den_ref)

    adj = adj_ref[...]
    col = j * bc + jax.lax.broadcasted_iota(jnp.int32, (1, bc), 1)
    mask = (adj > 0) & (col < n_valid)
    f1 = f1_ref[...]
    m = m_ref[...]

    for h in range(nheads):
        f2h = f2t_ref[h:h + 1, pl.ds(j * bc, bc)]
        e = _leaky(f1[:, h:h + 1] + f2h)
        p = jnp.where(mask, jnp.exp(e - m[:, h:h + 1]), 0.0)
        den_ref[:, h:h + 1] += jnp.sum(p, axis=1, keepdims=True)
        whh = wh_ref[pl.ds(j * bc, bc), h * od:(h + 1) * od]
        out_ref[:, h * od:(h + 1) * od] += jnp.dot(
            p, whh, preferred_element_type=jnp.float32)

    @pl.when(j == nj - 1)
    def _finish():
        acc = out_ref[...]
        den = den_ref[...]
        cols = []
        for h in range(nheads):
            d = den[:, h:h + 1]
            hp = jnp.where(d > 0, acc[:, h * od:(h + 1) * od] / d,
                           whmean_ref[0:1, h * od:(h + 1) * od])
            cols.append(jnp.where(hp > 0, hp, jnp.exp(hp) - 1.0))
        out_ref[...] = jnp.concatenate(cols, axis=1)


def kernel(x, adj, W, a):
    n, in_dim = x.shape
    nheads, _, od = W.shape
    ho = nheads * od

    bm = 256
    npad0 = pl.cdiv(n, bm) * bm
    bc = min(2048, npad0)
    nj = pl.cdiv(npad0, bc)
    npad = nj * bc
    ni = npad // bm

    xp = jnp.pad(x, ((0, npad - n), (0, 0)))

    wcat = jnp.concatenate([W[h] for h in range(nheads)], axis=1)
    a1 = jnp.zeros((ho, nheads), jnp.float32)
    a2 = jnp.zeros((ho, nheads), jnp.float32)
    for h in range(nheads):
        a1 = a1.at[h * od:(h + 1) * od, h].set(a[h, :od, 0])
        a2 = a2.at[h * od:(h + 1) * od, h].set(a[h, od:, 0])

    wh, f1, f2 = pl.pallas_call(
        functools.partial(_proj_kernel, n_valid=n, bm=bm),
        grid=(ni,),
        in_specs=[
            pl.BlockSpec((bm, in_dim), lambda i: (i, 0)),
            pl.BlockSpec((in_dim, ho), lambda i: (0, 0)),
            pl.BlockSpec((ho, nheads), lambda i: (0, 0)),
            pl.BlockSpec((ho, nheads), lambda i: (0, 0)),
        ],
        out_specs=[
            pl.BlockSpec((bm, ho), lambda i: (i, 0)),
            pl.BlockSpec((bm, nheads), lambda i: (i, 0)),
            pl.BlockSpec((bm, nheads), lambda i: (i, 0)),
        ],
        out_shape=[
            jax.ShapeDtypeStruct((npad, ho), jnp.float32),
            jax.ShapeDtypeStruct((npad, nheads), jnp.float32),
            jax.ShapeDtypeStruct((npad, nheads), jnp.float32),
        ],
        compiler_params=pltpu.CompilerParams(
            dimension_semantics=("arbitrary",)),
    )(xp, wcat, a1, a2)

    gmax = jnp.max(f2[:n], axis=0, keepdims=True)
    m = _leaky(f1 + gmax)
    whmean = jnp.sum(wh, axis=0, keepdims=True) / n
    f2t = f2.T

    out = pl.pallas_call(
        functools.partial(_att_kernel, n_valid=n, bc=bc, nj=nj,
                          nheads=nheads, od=od),
        grid=(ni, nj),
        in_specs=[
            pl.BlockSpec((bm, nheads), lambda i, j: (i, 0)),
            pl.BlockSpec((bm, nheads), lambda i, j: (i, 0)),
            pl.BlockSpec((nheads, npad), lambda i, j: (0, 0)),
            pl.BlockSpec((1, ho), lambda i, j: (0, 0)),
            pl.BlockSpec((bm, bc), lambda i, j: (i, j)),
            pl.BlockSpec((npad, ho), lambda i, j: (0, 0)),
        ],
        out_specs=pl.BlockSpec((bm, ho), lambda i, j: (i, 0)),
        out_shape=jax.ShapeDtypeStruct((n, ho), jnp.float32),
        scratch_shapes=[pltpu.VMEM((bm, nheads), jnp.float32)],
        compiler_params=pltpu.CompilerParams(
            dimension_semantics=("parallel", "arbitrary")),
    )(f1, m, f2t, whmean, adj, wh)

    return out


# speedup vs baseline: 2.8184x; 1.4782x over previous
import functools

import jax
import jax.numpy as jnp
from jax.experimental import pallas as pl
from jax.experimental.pallas import tpu as pltpu

ALPHA = 0.2
LOG2E = 1.4426950408889634
NEG_BIG = -1e30


def _proj_kernel(x_ref, wcat_ref, a1_ref, a2_ref, e_ref, ones_ref,
                 whaug_ref, f1_ref, f2_ref):
    wh = jnp.dot(x_ref[...], wcat_ref[...], preferred_element_type=jnp.float32)
    f1_ref[...] = jnp.dot(wh, a1_ref[...], preferred_element_type=jnp.float32)
    f2_ref[...] = jnp.dot(wh, a2_ref[...], preferred_element_type=jnp.float32)
    aug = jnp.dot(wh, e_ref[...], preferred_element_type=jnp.float32)
    whaug_ref[...] = (aug + ones_ref[...]).astype(jnp.bfloat16)


def _att_kernel(b1_ref, b2_ref, f2l_ref, c2l_ref, whmean_ref, adj_ref,
                whaug_ref, out_ref, acc_ref, *, bc, nj, nheads, od):
    j = pl.program_id(1)

    @pl.when(j == 0)
    def _init():
        acc_ref[...] = jnp.zeros_like(acc_ref)

    fm = adj_ref[...].astype(jnp.float32)
    b1 = b1_ref[...]
    b2 = b2_ref[...]

    for h in range(nheads):
        t1 = b1[:, h:h + 1] + f2l_ref[h:h + 1, pl.ds(j * bc, bc)]
        t2 = b2[:, h:h + 1] + c2l_ref[h:h + 1, pl.ds(j * bc, bc)]
        p = jnp.exp2(jnp.maximum(t1, t2)) * fm
        pb = p.astype(jnp.bfloat16)
        wa = whaug_ref[pl.ds(j * bc, bc), h * 128:(h + 1) * 128]
        acc_ref[:, h * 128:(h + 1) * 128] += jnp.dot(
            pb, wa, preferred_element_type=jnp.float32)

    @pl.when(j == nj - 1)
    def _finish():
        acc = acc_ref[...]
        cols = []
        for h in range(nheads):
            d = acc[:, h * 128 + od:h * 128 + od + 1]
            num = acc[:, h * 128:h * 128 + od]
            hp = jnp.where(d > 0, num / d,
                           whmean_ref[0:1, h * od:(h + 1) * od])
            cols.append(jnp.where(hp > 0, hp, jnp.exp(hp) - 1.0))
        out_ref[...] = jnp.concatenate(cols, axis=1)


def kernel(x, adj, W, a):
    n, in_dim = x.shape
    nheads, _, od = W.shape
    ho = nheads * od

    bm = 256
    npad0 = pl.cdiv(n, bm) * bm
    bc = min(2048, npad0)
    nj = pl.cdiv(npad0, bc)
    npad = nj * bc
    ni = npad // bm

    xp = jnp.pad(x, ((0, npad - n), (0, 0)))

    wcat = jnp.concatenate([W[h] for h in range(nheads)], axis=1)
    a1 = jnp.zeros((ho, nheads), jnp.float32)
    a2 = jnp.zeros((ho, nheads), jnp.float32)
    for h in range(nheads):
        a1 = a1.at[h * od:(h + 1) * od, h].set(a[h, :od, 0])
        a2 = a2.at[h * od:(h + 1) * od, h].set(a[h, od:, 0])
    emat = jnp.zeros((ho, nheads * 128), jnp.float32)
    onesrow = jnp.zeros((1, nheads * 128), jnp.float32)
    for h in range(nheads):
        emat = emat.at[h * od:(h + 1) * od, h * 128:h * 128 + od].set(
            jnp.eye(od, dtype=jnp.float32))
        onesrow = onesrow.at[0, h * 128 + od].set(1.0)

    whaug, f1, f2 = pl.pallas_call(
        _proj_kernel,
        grid=(ni,),
        in_specs=[
            pl.BlockSpec((bm, in_dim), lambda i: (i, 0)),
            pl.BlockSpec((in_dim, ho), lambda i: (0, 0)),
            pl.BlockSpec((ho, nheads), lambda i: (0, 0)),
            pl.BlockSpec((ho, nheads), lambda i: (0, 0)),
            pl.BlockSpec((ho, nheads * 128), lambda i: (0, 0)),
            pl.BlockSpec((1, nheads * 128), lambda i: (0, 0)),
        ],
        out_specs=[
            pl.BlockSpec((bm, nheads * 128), lambda i: (i, 0)),
            pl.BlockSpec((bm, nheads), lambda i: (i, 0)),
            pl.BlockSpec((bm, nheads), lambda i: (i, 0)),
        ],
        out_shape=[
            jax.ShapeDtypeStruct((npad, nheads * 128), jnp.bfloat16),
            jax.ShapeDtypeStruct((npad, nheads), jnp.float32),
            jax.ShapeDtypeStruct((npad, nheads), jnp.float32),
        ],
        compiler_params=pltpu.CompilerParams(
            dimension_semantics=("arbitrary",)),
    )(xp, wcat, a1, a2, emat, onesrow)

    gmax = jnp.max(f2[:n], axis=0, keepdims=True)
    m = jnp.maximum(f1 + gmax, ALPHA * (f1 + gmax))
    b1 = (f1 - m) * LOG2E
    b2 = (ALPHA * f1 - m) * LOG2E
    colpad = jnp.arange(npad)[:, None] >= n
    f2k = jnp.where(colpad, NEG_BIG, f2)
    f2l = (f2k * LOG2E).T
    c2l = (jnp.where(colpad, NEG_BIG, ALPHA * f2) * LOG2E).T
    xmean = jnp.mean(x, axis=0, keepdims=True)
    whmean = jnp.dot(xmean, wcat)

    out = pl.pallas_call(
        functools.partial(_att_kernel, bc=bc, nj=nj, nheads=nheads, od=od),
        grid=(ni, nj),
        in_specs=[
            pl.BlockSpec((bm, nheads), lambda i, j: (i, 0)),
            pl.BlockSpec((bm, nheads), lambda i, j: (i, 0)),
            pl.BlockSpec((nheads, npad), lambda i, j: (0, 0)),
            pl.BlockSpec((nheads, npad), lambda i, j: (0, 0)),
            pl.BlockSpec((1, ho), lambda i, j: (0, 0)),
            pl.BlockSpec((bm, bc), lambda i, j: (i, j)),
            pl.BlockSpec((npad, nheads * 128), lambda i, j: (0, 0)),
        ],
        out_specs=pl.BlockSpec((bm, ho), lambda i, j: (i, 0)),
        out_shape=jax.ShapeDtypeStruct((n, ho), jnp.float32),
        scratch_shapes=[pltpu.VMEM((bm, nheads * 128), jnp.float32)],
        compiler_params=pltpu.CompilerParams(
            dimension_semantics=("parallel", "arbitrary")),
    )(b1, b2, f2l, c2l, whmean, adj, whaug)

    return out


# speedup vs baseline: 2.9108x; 1.0328x over previous
import functools

import jax
import jax.numpy as jnp
from jax.experimental import pallas as pl
from jax.experimental.pallas import tpu as pltpu

ALPHA = 0.2
LOG2E = 1.4426950408889634
NEG_BIG = -1e30


def _proj_kernel(x_ref, wcat_ref, a1_ref, a2_ref, e_ref, ones_ref,
                 whaug_ref, f1_ref, f2_ref):
    wh = jnp.dot(x_ref[...], wcat_ref[...], preferred_element_type=jnp.float32)
    f1_ref[...] = jnp.dot(wh, a1_ref[...], preferred_element_type=jnp.float32)
    f2_ref[...] = jnp.dot(wh, a2_ref[...], preferred_element_type=jnp.float32)
    aug = jnp.dot(wh, e_ref[...], preferred_element_type=jnp.float32)
    whaug_ref[...] = (aug + ones_ref[...]).astype(jnp.bfloat16)


def _att_kernel(u1_ref, u2_ref, v1_ref, v2_ref, whmean_ref, adj_ref,
                whaug_ref, out_ref, acc_ref, *, bc, nj, nheads, od):
    j = pl.program_id(1)

    @pl.when(j == 0)
    def _init():
        acc_ref[...] = jnp.zeros_like(acc_ref)

    fm = adj_ref[...].astype(jnp.float32)
    u1 = u1_ref[...]
    u2 = u2_ref[...]

    for h in range(nheads):
        t1 = u1[:, h:h + 1] * v1_ref[h:h + 1, pl.ds(j * bc, bc)]
        t2 = u2[:, h:h + 1] * v2_ref[h:h + 1, pl.ds(j * bc, bc)]
        p = jnp.maximum(t1, t2) * fm
        pb = p.astype(jnp.bfloat16)
        wa = whaug_ref[pl.ds(j * bc, bc), h * 128:(h + 1) * 128]
        acc_ref[:, h * 128:(h + 1) * 128] += jnp.dot(
            pb, wa, preferred_element_type=jnp.float32)

    @pl.when(j == nj - 1)
    def _finish():
        acc = acc_ref[...]
        cols = []
        for h in range(nheads):
            d = acc[:, h * 128 + od:h * 128 + od + 1]
            num = acc[:, h * 128:h * 128 + od]
            hp = jnp.where(d > 0, num / d,
                           whmean_ref[0:1, h * od:(h + 1) * od])
            cols.append(jnp.where(hp > 0, hp, jnp.exp(hp) - 1.0))
        out_ref[...] = jnp.concatenate(cols, axis=1)


def kernel(x, adj, W, a):
    n, in_dim = x.shape
    nheads, _, od = W.shape
    ho = nheads * od

    bm = 256
    npad0 = pl.cdiv(n, bm) * bm
    bc = min(2048, npad0)
    nj = pl.cdiv(npad0, bc)
    npad = nj * bc
    ni = npad // bm

    xp = jnp.pad(x, ((0, npad - n), (0, 0)))

    wcat = jnp.concatenate([W[h] for h in range(nheads)], axis=1)
    a1 = jnp.zeros((ho, nheads), jnp.float32)
    a2 = jnp.zeros((ho, nheads), jnp.float32)
    for h in range(nheads):
        a1 = a1.at[h * od:(h + 1) * od, h].set(a[h, :od, 0])
        a2 = a2.at[h * od:(h + 1) * od, h].set(a[h, od:, 0])
    emat = jnp.zeros((ho, nheads * 128), jnp.float32)
    onesrow = jnp.zeros((1, nheads * 128), jnp.float32)
    for h in range(nheads):
        emat = emat.at[h * od:(h + 1) * od, h * 128:h * 128 + od].set(
            jnp.eye(od, dtype=jnp.float32))
        onesrow = onesrow.at[0, h * 128 + od].set(1.0)

    whaug, f1, f2 = pl.pallas_call(
        _proj_kernel,
        grid=(ni,),
        in_specs=[
            pl.BlockSpec((bm, in_dim), lambda i: (i, 0)),
            pl.BlockSpec((in_dim, ho), lambda i: (0, 0)),
            pl.BlockSpec((ho, nheads), lambda i: (0, 0)),
            pl.BlockSpec((ho, nheads), lambda i: (0, 0)),
            pl.BlockSpec((ho, nheads * 128), lambda i: (0, 0)),
            pl.BlockSpec((1, nheads * 128), lambda i: (0, 0)),
        ],
        out_specs=[
            pl.BlockSpec((bm, nheads * 128), lambda i: (i, 0)),
            pl.BlockSpec((bm, nheads), lambda i: (i, 0)),
            pl.BlockSpec((bm, nheads), lambda i: (i, 0)),
        ],
        out_shape=[
            jax.ShapeDtypeStruct((npad, nheads * 128), jnp.bfloat16),
            jax.ShapeDtypeStruct((npad, nheads), jnp.float32),
            jax.ShapeDtypeStruct((npad, nheads), jnp.float32),
        ],
        compiler_params=pltpu.CompilerParams(
            dimension_semantics=("arbitrary",)),
    )(xp, wcat, a1, a2, emat, onesrow)

    gmax = jnp.max(f2[:n], axis=0, keepdims=True)
    m = jnp.maximum(f1 + gmax, ALPHA * (f1 + gmax))
    u1 = jnp.exp(f1 - m)
    u2 = jnp.exp(ALPHA * f1 - m)
    colpad = jnp.arange(npad)[:, None] >= n
    v1 = jnp.where(colpad, 0.0, jnp.exp(f2)).T
    v2 = jnp.where(colpad, 0.0, jnp.exp(ALPHA * f2)).T
    xmean = jnp.mean(x, axis=0, keepdims=True)
    whmean = jnp.dot(xmean, wcat)

    out = pl.pallas_call(
        functools.partial(_att_kernel, bc=bc, nj=nj, nheads=nheads, od=od),
        grid=(ni, nj),
        in_specs=[
            pl.BlockSpec((bm, nheads), lambda i, j: (i, 0)),
            pl.BlockSpec((bm, nheads), lambda i, j: (i, 0)),
            pl.BlockSpec((nheads, npad), lambda i, j: (0, 0)),
            pl.BlockSpec((nheads, npad), lambda i, j: (0, 0)),
            pl.BlockSpec((1, ho), lambda i, j: (0, 0)),
            pl.BlockSpec((bm, bc), lambda i, j: (i, j)),
            pl.BlockSpec((npad, nheads * 128), lambda i, j: (0, 0)),
        ],
        out_specs=pl.BlockSpec((bm, ho), lambda i, j: (i, 0)),
        out_shape=jax.ShapeDtypeStruct((n, ho), jnp.float32),
        scratch_shapes=[pltpu.VMEM((bm, nheads * 128), jnp.float32)],
        compiler_params=pltpu.CompilerParams(
            dimension_semantics=("parallel", "arbitrary")),
    )(u1, u2, v1, v2, whmean, adj, whaug)

    return out
